# baseline (speedup 1.0000x reference)
import jax
import jax.numpy as jnp
from jax import lax
from jax.experimental import pallas as pl
from jax.experimental.pallas import tpu as pltpu
from jax.experimental.pallas import tpu_sc as plsc

E = 8
D_MODEL = 1024
D_FF = 2048
T = 2048
BK = 128
BK_BITS = 7
C = 5120
NB = C // BK
NBP = 48
TB = 256
NSTEP = T // TB
NC, NS, L = 2, 16, 16
NW = NC * NS
TPW = T // NW


def _router_body(x_ref, rw_ref, w1x_ref, w2x_ref, p1_ref, p2_ref, bexp_ref,
                 acc_ref, e1s, e2s, r1s, r2s):
    i = pl.program_id(0)

    @pl.when(i == 0)
    def _():
        acc_ref[...] = jnp.zeros_like(acc_ref)

    @pl.when(i < NSTEP)
    def _():
        logits = lax.dot_general(
            x_ref[...], rw_ref[...], (((1,), (1,)), ((), ())),
            preferred_element_type=jnp.float32)
        lane8 = lax.broadcasted_iota(jnp.int32, (TB, E), 1)
        l1 = jnp.max(logits, axis=-1, keepdims=True)
        i1 = jnp.min(jnp.where(logits == l1, lane8, E), axis=-1,
                     keepdims=True)
        neg = jnp.where(lane8 == i1, -jnp.inf, logits)
        l2 = jnp.max(neg, axis=-1, keepdims=True)
        i2 = jnp.min(jnp.where(neg == l2, lane8, E), axis=-1, keepdims=True)
        w1 = 1.0 / (1.0 + jnp.exp(l2 - l1))

        lane16 = lax.broadcasted_iota(jnp.int32, (TB, 16), 1)
        mask = ((lane16 == i1) | (lane16 == i2)).astype(jnp.float32)

        row = lax.broadcasted_iota(jnp.int32, (TB, TB), 0)
        col = lax.broadcasted_iota(jnp.int32, (TB, TB), 1)
        tri = (col < row).astype(jnp.float32)
        ranks = acc_ref[...] + lax.dot_general(
            tri, mask, (((1,), (0,)), ((), ())),
            preferred_element_type=jnp.float32)
        acc_ref[...] = acc_ref[...] + jnp.sum(mask, axis=0, keepdims=True)

        r8 = ranks[:, :E]
        m1 = (lane8 == i1).astype(jnp.float32)
        m2 = (lane8 == i2).astype(jnp.float32)
        r1 = jnp.sum(m1 * r8, axis=-1, keepdims=True).astype(jnp.int32)
        r2 = jnp.sum(m2 * r8, axis=-1, keepdims=True).astype(jnp.int32)

        zerow = jnp.zeros((TB, 128), jnp.float32)
        w1x_ref[...] = zerow + w1
        w2x_ref[...] = zerow + (1.0 - w1)

        sl = pl.ds(i * TB, TB)
        e1s[sl, :] = i1
        e2s[sl, :] = i2
        r1s[sl, :] = r1
        r2s[sl, :] = r2

    @pl.when(i == NSTEP)
    def _():
        cnt = acc_ref[...].astype(jnp.int32)
        padded = ((cnt + (BK - 1)) >> BK_BITS) << BK_BITS
        lane16 = lax.broadcasted_iota(jnp.int32, (1, 16), 1)
        gs = jnp.zeros((1, 16), jnp.int32)
        for e in range(E - 1):
            gs = gs + jnp.where(lane16 > e, padded[:, e:e + 1],
                                jnp.int32(0))

        lane_t = lax.broadcasted_iota(jnp.int32, (T, 16), 1)
        gs_t = gs + jnp.zeros((T, 16), jnp.int32)
        g1 = jnp.sum(jnp.where(lane_t == e1s[...], gs_t, jnp.int32(0)),
                     axis=-1, keepdims=True)
        g2 = jnp.sum(jnp.where(lane_t == e2s[...], gs_t, jnp.int32(0)),
                     axis=-1, keepdims=True)
        p1_ref[...] = g1 + r1s[...]
        p2_ref[...] = g2 + r2s[...]

        bb = lax.broadcasted_iota(jnp.int32, (1, NBP), 1) * BK
        acc = jnp.zeros((1, NBP), jnp.int32)
        for e in range(E):
            acc = acc + jnp.where(bb >= gs[:, e:e + 1], jnp.int32(1),
                                  jnp.int32(0))
        bexp_ref[...] = acc - 1


def _router(x_flat, router_w):
    out_shapes = (
        jax.ShapeDtypeStruct((T, 128), jnp.float32),
        jax.ShapeDtypeStruct((T, 128), jnp.float32),
        jax.ShapeDtypeStruct((T, 1), jnp.int32),
        jax.ShapeDtypeStruct((T, 1), jnp.int32),
        jax.ShapeDtypeStruct((1, NBP), jnp.int32),
    )
    return pl.pallas_call(
        _router_body,
        grid=(NSTEP + 1,),
        in_specs=[
            pl.BlockSpec((TB, D_MODEL),
                         lambda i: (jnp.minimum(i, NSTEP - 1), 0)),
            pl.BlockSpec((E, D_MODEL), lambda i: (0, 0)),
        ],
        out_specs=(
            pl.BlockSpec((TB, 128), lambda i: (jnp.minimum(i, NSTEP - 1), 0)),
            pl.BlockSpec((TB, 128), lambda i: (jnp.minimum(i, NSTEP - 1), 0)),
            pl.BlockSpec((T, 1), lambda i: (0, 0)),
            pl.BlockSpec((T, 1), lambda i: (0, 0)),
            pl.BlockSpec((1, NBP), lambda i: (0, 0)),
        ),
        out_shape=out_shapes,
        scratch_shapes=[
            pltpu.VMEM((1, 16), jnp.float32),
            pltpu.VMEM((T, 1), jnp.int32),
            pltpu.VMEM((T, 1), jnp.int32),
            pltpu.VMEM((T, 1), jnp.int32),
            pltpu.VMEM((T, 1), jnp.int32),
        ],
        compiler_params=pltpu.CompilerParams(
            dimension_semantics=("arbitrary",)),
    )(x_flat, router_w)


def _dispatch_body(x_hbm, w1x_hbm, w2x_hbm, p1_hbm, p2_hbm,
                   xg_hbm, wrow_hbm,
                   xbuf, wb1, wb2, p1v, p2v, sem):
    wid = lax.axis_index("s") * NC + lax.axis_index("c")
    base = wid * TPW

    pltpu.sync_copy(x_hbm.at[pl.ds(base, TPW)], xbuf)
    pltpu.sync_copy(w1x_hbm.at[pl.ds(base, TPW)], wb1)
    pltpu.sync_copy(w2x_hbm.at[pl.ds(base, TPW)], wb2)
    pltpu.sync_copy(p1_hbm.at[pl.ds(base, TPW)], p1v)
    pltpu.sync_copy(p2_hbm.at[pl.ds(base, TPW)], p2v)

    d1 = pltpu.async_copy(xbuf, xg_hbm.at[p1v], sem)
    d2 = pltpu.async_copy(xbuf, xg_hbm.at[p2v], sem)
    d3 = pltpu.async_copy(wb1, wrow_hbm.at[p1v], sem)
    d4 = pltpu.async_copy(wb2, wrow_hbm.at[p2v], sem)
    d1.wait()
    d2.wait()
    d3.wait()
    d4.wait()


def _dispatch(x_flat, w1x, w2x, p1, p2):
    mesh = plsc.VectorSubcoreMesh(
        core_axis_name="c", subcore_axis_name="s",
        num_cores=NC, num_subcores=NS)
    out_type = (
        jax.ShapeDtypeStruct((C, D_MODEL), jnp.float32),
        jax.ShapeDtypeStruct((C, 128), jnp.float32),
    )
    f = pl.kernel(
        _dispatch_body,
        out_type=out_type,
        mesh=mesh,
        scratch_types=[
            pltpu.VMEM((TPW, D_MODEL), jnp.float32),
            pltpu.VMEM((TPW, 128), jnp.float32),
            pltpu.VMEM((TPW, 128), jnp.float32),
            pltpu.VMEM((TPW,), jnp.int32),
            pltpu.VMEM((TPW,), jnp.int32),
            pltpu.SemaphoreType.DMA,
        ],
    )
    return f(x_flat, w1x, w2x, p1, p2)


def _mm_body(bexp_ref, xg_ref, gw_ref, uw_ref, dw_ref, wr_ref, y_ref):
    xb = xg_ref[...].astype(jnp.bfloat16)
    g = lax.dot_general(xb, gw_ref[0], (((1,), (1,)), ((), ())),
                        preferred_element_type=jnp.float32)
    u = lax.dot_general(xb, uw_ref[0], (((1,), (1,)), ((), ())),
                        preferred_element_type=jnp.float32)
    h = (g * jax.nn.sigmoid(g) * u).astype(jnp.bfloat16)
    y = lax.dot_general(h, dw_ref[0], (((1,), (1,)), ((), ())),
                        preferred_element_type=jnp.float32)
    y_ref[...] = y * wr_ref[:, :1]


def _grouped_mm(bexp, xg, gate_w, up_w, down_w, wrow):
    grid_spec = pltpu.PrefetchScalarGridSpec(
        num_scalar_prefetch=1,
        grid=(NB,),
        in_specs=[
            pl.BlockSpec((BK, D_MODEL), lambda b, s: (b, 0)),
            pl.BlockSpec((1, D_FF, D_MODEL), lambda b, s: (s[b], 0, 0)),
            pl.BlockSpec((1, D_FF, D_MODEL), lambda b, s: (s[b], 0, 0)),
            pl.BlockSpec((1, D_MODEL, D_FF), lambda b, s: (s[b], 0, 0)),
            pl.BlockSpec((BK, 128), lambda b, s: (b, 0)),
        ],
        out_specs=pl.BlockSpec((BK, D_MODEL), lambda b, s: (b, 0)),
    )
    return pl.pallas_call(
        _mm_body,
        grid_spec=grid_spec,
        out_shape=jax.ShapeDtypeStruct((C, D_MODEL), jnp.float32),
        compiler_params=pltpu.CompilerParams(
            dimension_semantics=("arbitrary",)),
    )(bexp, xg, gate_w, up_w, down_w, wrow)


CHUNK = 32


def _combine_body(y_hbm, p1_hbm, p2_hbm, out_hbm,
                  p1v, p2v, buf1, buf2, sem):
    wid = lax.axis_index("s") * NC + lax.axis_index("c")
    base = wid * TPW

    for ci in range(TPW // CHUNK):
        b2 = base + ci * CHUNK
        pltpu.sync_copy(p1_hbm.at[pl.ds(b2, CHUNK)], p1v)
        pltpu.sync_copy(p2_hbm.at[pl.ds(b2, CHUNK)], p2v)
        d1 = pltpu.async_copy(y_hbm.at[p1v], buf1, sem)
        d2 = pltpu.async_copy(y_hbm.at[p2v], buf2, sem)
        d1.wait()
        d2.wait()

        def row_body(r, _):
            def col_body(c, _):
                sl = pl.ds(c * L, L)
                buf1[r, sl] = buf1[r, sl] + buf2[r, sl]
                return 0

            lax.fori_loop(0, D_MODEL // L, col_body, 0)
            return 0

        lax.fori_loop(0, CHUNK, row_body, 0)
        pltpu.sync_copy(buf1, out_hbm.at[pl.ds(b2, CHUNK)])


def _combine(y, p1, p2):
    mesh = plsc.VectorSubcoreMesh(
        core_axis_name="c", subcore_axis_name="s",
        num_cores=NC, num_subcores=NS)
    f = pl.kernel(
        _combine_body,
        out_type=jax.ShapeDtypeStruct((T, D_MODEL), jnp.float32),
        mesh=mesh,
        scratch_types=[
            pltpu.VMEM((CHUNK,), jnp.int32),
            pltpu.VMEM((CHUNK,), jnp.int32),
            pltpu.VMEM((CHUNK, D_MODEL), jnp.float32),
            pltpu.VMEM((CHUNK, D_MODEL), jnp.float32),
            pltpu.SemaphoreType.DMA,
        ],
    )
    return f(y, p1, p2)


def kernel(x, router_w, gate_w, up_w, down_w):
    shape = x.shape
    x_flat = x.reshape(-1, shape[-1]).astype(jnp.float32)

    w1x, w2x, p1, p2, bexp = _router(x_flat, router_w)
    p1 = p1.reshape(-1)
    p2 = p2.reshape(-1)
    xg, wrow = _dispatch(x_flat, w1x, w2x, p1, p2)
    y = _grouped_mm(bexp.reshape(-1)[:NB], xg,
                    gate_w.astype(jnp.bfloat16), up_w.astype(jnp.bfloat16),
                    down_w.astype(jnp.bfloat16), wrow)
    out = _combine(y, p1, p2)
    return out.reshape(shape)


# speedup vs baseline: 1.6675x; 1.6675x over previous
import jax
import jax.numpy as jnp
from jax import lax
from jax.experimental import pallas as pl
from jax.experimental.pallas import tpu as pltpu
from jax.experimental.pallas import tpu_sc as plsc

E = 8
D_MODEL = 1024
D_FF = 2048
T = 2048
BK = 256
BK_BITS = 8
C = 6144
NB = C // BK
NBP = 32
TB = 256
NSTEP = T // TB
NC, NS, L = 2, 16, 16
NW = NC * NS
TPW = T // NW


def _router_body(x_ref, rw_ref, w1x_ref, w2x_ref, p1_ref, p2_ref, bexp_ref,
                 acc_ref, e1s, e2s, r1s, r2s):
    i = pl.program_id(0)

    @pl.when(i == 0)
    def _():
        acc_ref[...] = jnp.zeros_like(acc_ref)

    @pl.when(i < NSTEP)
    def _():
        logits = lax.dot_general(
            x_ref[...], rw_ref[...], (((1,), (1,)), ((), ())),
            preferred_element_type=jnp.float32)
        lane8 = lax.broadcasted_iota(jnp.int32, (TB, E), 1)
        l1 = jnp.max(logits, axis=-1, keepdims=True)
        i1 = jnp.min(jnp.where(logits == l1, lane8, E), axis=-1,
                     keepdims=True)
        neg = jnp.where(lane8 == i1, -jnp.inf, logits)
        l2 = jnp.max(neg, axis=-1, keepdims=True)
        i2 = jnp.min(jnp.where(neg == l2, lane8, E), axis=-1, keepdims=True)
        w1 = 1.0 / (1.0 + jnp.exp(l2 - l1))

        lane16 = lax.broadcasted_iota(jnp.int32, (TB, 16), 1)
        mask = ((lane16 == i1) | (lane16 == i2)).astype(jnp.float32)

        row = lax.broadcasted_iota(jnp.int32, (TB, TB), 0)
        col = lax.broadcasted_iota(jnp.int32, (TB, TB), 1)
        tri = (col < row).astype(jnp.float32)
        ranks = acc_ref[...] + lax.dot_general(
            tri, mask, (((1,), (0,)), ((), ())),
            preferred_element_type=jnp.float32)
        acc_ref[...] = acc_ref[...] + jnp.sum(mask, axis=0, keepdims=True)

        r8 = ranks[:, :E]
        m1 = (lane8 == i1).astype(jnp.float32)
        m2 = (lane8 == i2).astype(jnp.float32)
        r1 = jnp.sum(m1 * r8, axis=-1, keepdims=True).astype(jnp.int32)
        r2 = jnp.sum(m2 * r8, axis=-1, keepdims=True).astype(jnp.int32)

        zerow = jnp.zeros((TB, 128), jnp.float32)
        w1x_ref[...] = zerow + w1
        w2x_ref[...] = zerow + (1.0 - w1)

        sl = pl.ds(i * TB, TB)
        e1s[sl, :] = i1
        e2s[sl, :] = i2
        r1s[sl, :] = r1
        r2s[sl, :] = r2

    @pl.when(i == NSTEP)
    def _():
        cnt = acc_ref[...].astype(jnp.int32)
        padded = ((cnt + (BK - 1)) >> BK_BITS) << BK_BITS
        lane16 = lax.broadcasted_iota(jnp.int32, (1, 16), 1)
        gs = jnp.zeros((1, 16), jnp.int32)
        for e in range(E - 1):
            gs = gs + jnp.where(lane16 > e, padded[:, e:e + 1],
                                jnp.int32(0))

        lane_t = lax.broadcasted_iota(jnp.int32, (T, 16), 1)
        gs_t = gs + jnp.zeros((T, 16), jnp.int32)
        g1 = jnp.sum(jnp.where(lane_t == e1s[...], gs_t, jnp.int32(0)),
                     axis=-1, keepdims=True)
        g2 = jnp.sum(jnp.where(lane_t == e2s[...], gs_t, jnp.int32(0)),
                     axis=-1, keepdims=True)
        p1_ref[...] = g1 + r1s[...]
        p2_ref[...] = g2 + r2s[...]

        bb = lax.broadcasted_iota(jnp.int32, (1, NBP), 1) * BK
        acc = jnp.zeros((1, NBP), jnp.int32)
        for e in range(E):
            acc = acc + jnp.where(bb >= gs[:, e:e + 1], jnp.int32(1),
                                  jnp.int32(0))
        bexp_ref[...] = acc - 1


def _router(x_flat, router_w):
    out_shapes = (
        jax.ShapeDtypeStruct((T, 128), jnp.float32),
        jax.ShapeDtypeStruct((T, 128), jnp.float32),
        jax.ShapeDtypeStruct((T, 1), jnp.int32),
        jax.ShapeDtypeStruct((T, 1), jnp.int32),
        jax.ShapeDtypeStruct((1, NBP), jnp.int32),
    )
    return pl.pallas_call(
        _router_body,
        grid=(NSTEP + 1,),
        in_specs=[
            pl.BlockSpec((TB, D_MODEL),
                         lambda i: (jnp.minimum(i, NSTEP - 1), 0)),
            pl.BlockSpec((E, D_MODEL), lambda i: (0, 0)),
        ],
        out_specs=(
            pl.BlockSpec((TB, 128), lambda i: (jnp.minimum(i, NSTEP - 1), 0)),
            pl.BlockSpec((TB, 128), lambda i: (jnp.minimum(i, NSTEP - 1), 0)),
            pl.BlockSpec((T, 1), lambda i: (0, 0)),
            pl.BlockSpec((T, 1), lambda i: (0, 0)),
            pl.BlockSpec((1, NBP), lambda i: (0, 0)),
        ),
        out_shape=out_shapes,
        scratch_shapes=[
            pltpu.VMEM((1, 16), jnp.float32),
            pltpu.VMEM((T, 1), jnp.int32),
            pltpu.VMEM((T, 1), jnp.int32),
            pltpu.VMEM((T, 1), jnp.int32),
            pltpu.VMEM((T, 1), jnp.int32),
        ],
        compiler_params=pltpu.CompilerParams(
            dimension_semantics=("arbitrary",)),
    )(x_flat, router_w)


def _dispatch_body(x_hbm, w1x_hbm, w2x_hbm, p1_hbm, p2_hbm,
                   xg_hbm, wrow_hbm,
                   xbuf, wb1, wb2, p1v, p2v, sem):
    wid = lax.axis_index("s") * NC + lax.axis_index("c")
    base = wid * TPW

    pltpu.sync_copy(x_hbm.at[pl.ds(base, TPW)], xbuf)
    pltpu.sync_copy(w1x_hbm.at[pl.ds(base, TPW)], wb1)
    pltpu.sync_copy(w2x_hbm.at[pl.ds(base, TPW)], wb2)
    pltpu.sync_copy(p1_hbm.at[pl.ds(base, TPW)], p1v)
    pltpu.sync_copy(p2_hbm.at[pl.ds(base, TPW)], p2v)

    d1 = pltpu.async_copy(xbuf, xg_hbm.at[p1v], sem)
    d2 = pltpu.async_copy(xbuf, xg_hbm.at[p2v], sem)
    d3 = pltpu.async_copy(wb1, wrow_hbm.at[p1v], sem)
    d4 = pltpu.async_copy(wb2, wrow_hbm.at[p2v], sem)
    d1.wait()
    d2.wait()
    d3.wait()
    d4.wait()


def _dispatch(x_flat, w1x, w2x, p1, p2):
    mesh = plsc.VectorSubcoreMesh(
        core_axis_name="c", subcore_axis_name="s",
        num_cores=NC, num_subcores=NS)
    out_type = (
        jax.ShapeDtypeStruct((C, D_MODEL), jnp.float32),
        jax.ShapeDtypeStruct((C, 128), jnp.float32),
    )
    f = pl.kernel(
        _dispatch_body,
        out_type=out_type,
        mesh=mesh,
        scratch_types=[
            pltpu.VMEM((TPW, D_MODEL), jnp.float32),
            pltpu.VMEM((TPW, 128), jnp.float32),
            pltpu.VMEM((TPW, 128), jnp.float32),
            pltpu.VMEM((TPW,), jnp.int32),
            pltpu.VMEM((TPW,), jnp.int32),
            pltpu.SemaphoreType.DMA,
        ],
    )
    return f(x_flat, w1x, w2x, p1, p2)


def _mm_body(bexp_ref, xg_ref, gw_ref, uw_ref, dw_ref, wr_ref, y_ref):
    xb = xg_ref[...]
    g = lax.dot_general(xb, gw_ref[0], (((1,), (1,)), ((), ())),
                        preferred_element_type=jnp.float32)
    u = lax.dot_general(xb, uw_ref[0], (((1,), (1,)), ((), ())),
                        preferred_element_type=jnp.float32)
    h = g * jax.nn.sigmoid(g) * u
    y = lax.dot_general(h, dw_ref[0], (((1,), (1,)), ((), ())),
                        preferred_element_type=jnp.float32)
    y_ref[...] = y * wr_ref[:, :1]


def _grouped_mm(bexp, xg, gate_w, up_w, down_w, wrow):
    grid_spec = pltpu.PrefetchScalarGridSpec(
        num_scalar_prefetch=1,
        grid=(NB,),
        in_specs=[
            pl.BlockSpec((BK, D_MODEL), lambda b, s: (b, 0)),
            pl.BlockSpec((1, D_FF, D_MODEL), lambda b, s: (s[b], 0, 0)),
            pl.BlockSpec((1, D_FF, D_MODEL), lambda b, s: (s[b], 0, 0)),
            pl.BlockSpec((1, D_MODEL, D_FF), lambda b, s: (s[b], 0, 0)),
            pl.BlockSpec((BK, 128), lambda b, s: (b, 0)),
        ],
        out_specs=pl.BlockSpec((BK, D_MODEL), lambda b, s: (b, 0)),
    )
    return pl.pallas_call(
        _mm_body,
        grid_spec=grid_spec,
        out_shape=jax.ShapeDtypeStruct((C, D_MODEL), jnp.float32),
        compiler_params=pltpu.CompilerParams(
            dimension_semantics=("arbitrary",)),
    )(bexp, xg, gate_w, up_w, down_w, wrow)


CHUNK = 32


def _combine_body(y_hbm, p1_hbm, p2_hbm, out_hbm,
                  p1v, p2v, buf1, buf2, sem):
    wid = lax.axis_index("s") * NC + lax.axis_index("c")
    base = wid * TPW

    for ci in range(TPW // CHUNK):
        b2 = base + ci * CHUNK
        pltpu.sync_copy(p1_hbm.at[pl.ds(b2, CHUNK)], p1v)
        pltpu.sync_copy(p2_hbm.at[pl.ds(b2, CHUNK)], p2v)
        d1 = pltpu.async_copy(y_hbm.at[p1v], buf1, sem)
        d2 = pltpu.async_copy(y_hbm.at[p2v], buf2, sem)
        d1.wait()
        d2.wait()

        def row_body(r, _):
            def col_body(c, _):
                sl = pl.ds(c * L, L)
                buf1[r, sl] = buf1[r, sl] + buf2[r, sl]
                return 0

            lax.fori_loop(0, D_MODEL // L, col_body, 0)
            return 0

        lax.fori_loop(0, CHUNK, row_body, 0)
        pltpu.sync_copy(buf1, out_hbm.at[pl.ds(b2, CHUNK)])


def _combine(y, p1, p2):
    mesh = plsc.VectorSubcoreMesh(
        core_axis_name="c", subcore_axis_name="s",
        num_cores=NC, num_subcores=NS)
    f = pl.kernel(
        _combine_body,
        out_type=jax.ShapeDtypeStruct((T, D_MODEL), jnp.float32),
        mesh=mesh,
        scratch_types=[
            pltpu.VMEM((CHUNK,), jnp.int32),
            pltpu.VMEM((CHUNK,), jnp.int32),
            pltpu.VMEM((CHUNK, D_MODEL), jnp.float32),
            pltpu.VMEM((CHUNK, D_MODEL), jnp.float32),
            pltpu.SemaphoreType.DMA,
        ],
    )
    return f(y, p1, p2)


def kernel(x, router_w, gate_w, up_w, down_w):
    shape = x.shape
    x_flat = x.reshape(-1, shape[-1]).astype(jnp.float32)

    w1x, w2x, p1, p2, bexp = _router(x_flat, router_w)
    p1 = p1.reshape(-1)
    p2 = p2.reshape(-1)
    xg, wrow = _dispatch(x_flat, w1x, w2x, p1, p2)
    y = _grouped_mm(bexp.reshape(-1)[:NB], xg, gate_w, up_w, down_w, wrow)
    out = _combine(y, p1, p2)
    return out.reshape(shape)


# speedup vs baseline: 1.8115x; 1.0864x over previous
import jax
import jax.numpy as jnp
from jax import lax
from jax.experimental import pallas as pl
from jax.experimental.pallas import tpu as pltpu
from jax.experimental.pallas import tpu_sc as plsc

E = 8
D_MODEL = 1024
D_FF = 2048
T = 2048
BK = 256
BK_BITS = 8
C = 6144
NB = C // BK
NBP = 32
TB = 256
NSTEP = T // TB
NC, NS, L = 2, 16, 16
NW = NC * NS
TPW = T // NW


def _router_body(x_ref, rw_ref, w1x_ref, w2x_ref, p1_ref, p2_ref, bexp_ref,
                 acc_ref, e1s, e2s, r1s, r2s):
    i = pl.program_id(0)

    @pl.when(i == 0)
    def _():
        acc_ref[...] = jnp.zeros_like(acc_ref)

    @pl.when(i < NSTEP)
    def _():
        logits = lax.dot_general(
            x_ref[...], rw_ref[...], (((1,), (1,)), ((), ())),
            preferred_element_type=jnp.float32)
        lane8 = lax.broadcasted_iota(jnp.int32, (TB, E), 1)
        l1 = jnp.max(logits, axis=-1, keepdims=True)
        i1 = jnp.min(jnp.where(logits == l1, lane8, E), axis=-1,
                     keepdims=True)
        neg = jnp.where(lane8 == i1, -jnp.inf, logits)
        l2 = jnp.max(neg, axis=-1, keepdims=True)
        i2 = jnp.min(jnp.where(neg == l2, lane8, E), axis=-1, keepdims=True)
        w1 = 1.0 / (1.0 + jnp.exp(l2 - l1))

        lane16 = lax.broadcasted_iota(jnp.int32, (TB, 16), 1)
        mask = ((lane16 == i1) | (lane16 == i2)).astype(jnp.float32)

        row = lax.broadcasted_iota(jnp.int32, (TB, TB), 0)
        col = lax.broadcasted_iota(jnp.int32, (TB, TB), 1)
        tri = (col < row).astype(jnp.float32)
        ranks = acc_ref[...] + lax.dot_general(
            tri, mask, (((1,), (0,)), ((), ())),
            preferred_element_type=jnp.float32)
        acc_ref[...] = acc_ref[...] + jnp.sum(mask, axis=0, keepdims=True)

        r8 = ranks[:, :E]
        m1 = (lane8 == i1).astype(jnp.float32)
        m2 = (lane8 == i2).astype(jnp.float32)
        r1 = jnp.sum(m1 * r8, axis=-1, keepdims=True).astype(jnp.int32)
        r2 = jnp.sum(m2 * r8, axis=-1, keepdims=True).astype(jnp.int32)

        zerow = jnp.zeros((TB, 128), jnp.float32)
        w1x_ref[...] = zerow + w1
        w2x_ref[...] = zerow + (1.0 - w1)

        sl = pl.ds(i * TB, TB)
        e1s[sl, :] = i1
        e2s[sl, :] = i2
        r1s[sl, :] = r1
        r2s[sl, :] = r2

    @pl.when(i == NSTEP)
    def _():
        cnt = acc_ref[...].astype(jnp.int32)
        padded = ((cnt + (BK - 1)) >> BK_BITS) << BK_BITS
        lane16 = lax.broadcasted_iota(jnp.int32, (1, 16), 1)
        gs = jnp.zeros((1, 16), jnp.int32)
        for e in range(E - 1):
            gs = gs + jnp.where(lane16 > e, padded[:, e:e + 1],
                                jnp.int32(0))

        lane_t = lax.broadcasted_iota(jnp.int32, (T, 16), 1)
        gs_t = gs + jnp.zeros((T, 16), jnp.int32)
        g1 = jnp.sum(jnp.where(lane_t == e1s[...], gs_t, jnp.int32(0)),
                     axis=-1, keepdims=True)
        g2 = jnp.sum(jnp.where(lane_t == e2s[...], gs_t, jnp.int32(0)),
                     axis=-1, keepdims=True)
        p1_ref[...] = g1 + r1s[...]
        p2_ref[...] = g2 + r2s[...]

        bb = lax.broadcasted_iota(jnp.int32, (1, NBP), 1) * BK
        acc = jnp.zeros((1, NBP), jnp.int32)
        for e in range(E):
            acc = acc + jnp.where(bb >= gs[:, e:e + 1], jnp.int32(1),
                                  jnp.int32(0))
        nact = jnp.sum(padded, axis=-1, keepdims=True) >> BK_BITS
        lanebp = lax.broadcasted_iota(jnp.int32, (1, NBP), 1)
        bexp_ref[...] = jnp.where(lanebp == NB, nact + jnp.zeros(
            (1, NBP), jnp.int32), acc - 1)


def _router(x_flat, router_w):
    out_shapes = (
        jax.ShapeDtypeStruct((T, 128), jnp.float32),
        jax.ShapeDtypeStruct((T, 128), jnp.float32),
        jax.ShapeDtypeStruct((T, 1), jnp.int32),
        jax.ShapeDtypeStruct((T, 1), jnp.int32),
        jax.ShapeDtypeStruct((1, NBP), jnp.int32),
    )
    return pl.pallas_call(
        _router_body,
        grid=(NSTEP + 1,),
        in_specs=[
            pl.BlockSpec((TB, D_MODEL),
                         lambda i: (jnp.minimum(i, NSTEP - 1), 0)),
            pl.BlockSpec((E, D_MODEL), lambda i: (0, 0)),
        ],
        out_specs=(
            pl.BlockSpec((TB, 128), lambda i: (jnp.minimum(i, NSTEP - 1), 0)),
            pl.BlockSpec((TB, 128), lambda i: (jnp.minimum(i, NSTEP - 1), 0)),
            pl.BlockSpec((T, 1), lambda i: (0, 0)),
            pl.BlockSpec((T, 1), lambda i: (0, 0)),
            pl.BlockSpec((1, NBP), lambda i: (0, 0)),
        ),
        out_shape=out_shapes,
        scratch_shapes=[
            pltpu.VMEM((1, 16), jnp.float32),
            pltpu.VMEM((T, 1), jnp.int32),
            pltpu.VMEM((T, 1), jnp.int32),
            pltpu.VMEM((T, 1), jnp.int32),
            pltpu.VMEM((T, 1), jnp.int32),
        ],
        compiler_params=pltpu.CompilerParams(
            dimension_semantics=("arbitrary",)),
    )(x_flat, router_w)


def _dispatch_body(x_hbm, w1x_hbm, w2x_hbm, p1_hbm, p2_hbm,
                   xg_hbm, wrow_hbm,
                   xbuf, wb1, wb2, p1v, p2v, sem):
    wid = lax.axis_index("s") * NC + lax.axis_index("c")
    base = wid * TPW

    pltpu.sync_copy(x_hbm.at[pl.ds(base, TPW)], xbuf)
    pltpu.sync_copy(w1x_hbm.at[pl.ds(base, TPW)], wb1)
    pltpu.sync_copy(w2x_hbm.at[pl.ds(base, TPW)], wb2)
    pltpu.sync_copy(p1_hbm.at[pl.ds(base, TPW)], p1v)
    pltpu.sync_copy(p2_hbm.at[pl.ds(base, TPW)], p2v)

    d1 = pltpu.async_copy(xbuf, xg_hbm.at[p1v], sem)
    d2 = pltpu.async_copy(xbuf, xg_hbm.at[p2v], sem)
    d3 = pltpu.async_copy(wb1, wrow_hbm.at[p1v], sem)
    d4 = pltpu.async_copy(wb2, wrow_hbm.at[p2v], sem)
    d1.wait()
    d2.wait()
    d3.wait()
    d4.wait()


def _dispatch(x_flat, w1x, w2x, p1, p2):
    mesh = plsc.VectorSubcoreMesh(
        core_axis_name="c", subcore_axis_name="s",
        num_cores=NC, num_subcores=NS)
    out_type = (
        jax.ShapeDtypeStruct((C, D_MODEL), jnp.float32),
        jax.ShapeDtypeStruct((C, 128), jnp.float32),
    )
    f = pl.kernel(
        _dispatch_body,
        out_type=out_type,
        mesh=mesh,
        scratch_types=[
            pltpu.VMEM((TPW, D_MODEL), jnp.float32),
            pltpu.VMEM((TPW, 128), jnp.float32),
            pltpu.VMEM((TPW, 128), jnp.float32),
            pltpu.VMEM((TPW,), jnp.int32),
            pltpu.VMEM((TPW,), jnp.int32),
            pltpu.SemaphoreType.DMA,
        ],
    )
    return f(x_flat, w1x, w2x, p1, p2)


def _mm_body(bexp_ref, xg_ref, gw_ref, uw_ref, dw_ref, wr_ref, y_ref):
    b = pl.program_id(0)

    @pl.when(b < bexp_ref[NB])
    def _():
        xb = xg_ref[...]
        g = lax.dot_general(xb, gw_ref[0], (((1,), (1,)), ((), ())),
                            preferred_element_type=jnp.float32)
        u = lax.dot_general(xb, uw_ref[0], (((1,), (1,)), ((), ())),
                            preferred_element_type=jnp.float32)
        h = g * jax.nn.sigmoid(g) * u
        y = lax.dot_general(h, dw_ref[0], (((1,), (1,)), ((), ())),
                            preferred_element_type=jnp.float32)
        y_ref[...] = y * wr_ref[:, :1]


def _grouped_mm(bexp, xg, gate_w, up_w, down_w, wrow):
    grid_spec = pltpu.PrefetchScalarGridSpec(
        num_scalar_prefetch=1,
        grid=(NB,),
        in_specs=[
            pl.BlockSpec((BK, D_MODEL), lambda b, s: (b, 0)),
            pl.BlockSpec((1, D_FF, D_MODEL), lambda b, s: (s[b], 0, 0)),
            pl.BlockSpec((1, D_FF, D_MODEL), lambda b, s: (s[b], 0, 0)),
            pl.BlockSpec((1, D_MODEL, D_FF), lambda b, s: (s[b], 0, 0)),
            pl.BlockSpec((BK, 128), lambda b, s: (b, 0)),
        ],
        out_specs=pl.BlockSpec((BK, D_MODEL), lambda b, s: (b, 0)),
    )
    return pl.pallas_call(
        _mm_body,
        grid_spec=grid_spec,
        out_shape=jax.ShapeDtypeStruct((C, D_MODEL), jnp.float32),
        compiler_params=pltpu.CompilerParams(
            dimension_semantics=("arbitrary",)),
    )(bexp, xg, gate_w, up_w, down_w, wrow)


CHUNK = 16
NCH = TPW // CHUNK


def _combine_body(y_hbm, p1_hbm, p2_hbm, out_hbm,
                  p1v, p2v, b1a, b2a, b1b, b2b, sema, semb):
    wid = lax.axis_index("s") * NC + lax.axis_index("c")
    base = wid * TPW

    pltpu.sync_copy(p1_hbm.at[pl.ds(base, TPW)], p1v)
    pltpu.sync_copy(p2_hbm.at[pl.ds(base, TPW)], p2v)

    b1 = (b1a, b1b)
    b2 = (b2a, b2b)
    sems = (sema, semb)

    def issue(ci):
        par = ci & 1
        sl = pl.ds(ci * CHUNK, CHUNK)
        d1 = pltpu.async_copy(y_hbm.at[p1v[sl]], b1[par], sems[par])
        d2 = pltpu.async_copy(y_hbm.at[p2v[sl]], b2[par], sems[par])
        return d1, d2

    cur = issue(0)
    for ci in range(NCH):
        par = ci & 1
        nxt = issue(ci + 1) if ci + 1 < NCH else None
        cur[0].wait()
        cur[1].wait()

        def row_body(r, _):
            def col_body(c, _):
                sl = pl.ds(c * L, L)
                b1[par][r, sl] = b1[par][r, sl] + b2[par][r, sl]
                return 0

            lax.fori_loop(0, D_MODEL // L, col_body, 0)
            return 0

        lax.fori_loop(0, CHUNK, row_body, 0)
        pltpu.sync_copy(b1[par], out_hbm.at[pl.ds(base + ci * CHUNK, CHUNK)])
        cur = nxt


def _combine(y, p1, p2):
    mesh = plsc.VectorSubcoreMesh(
        core_axis_name="c", subcore_axis_name="s",
        num_cores=NC, num_subcores=NS)
    f = pl.kernel(
        _combine_body,
        out_type=jax.ShapeDtypeStruct((T, D_MODEL), jnp.float32),
        mesh=mesh,
        scratch_types=[
            pltpu.VMEM((TPW,), jnp.int32),
            pltpu.VMEM((TPW,), jnp.int32),
            pltpu.VMEM((CHUNK, D_MODEL), jnp.float32),
            pltpu.VMEM((CHUNK, D_MODEL), jnp.float32),
            pltpu.VMEM((CHUNK, D_MODEL), jnp.float32),
            pltpu.VMEM((CHUNK, D_MODEL), jnp.float32),
            pltpu.SemaphoreType.DMA,
            pltpu.SemaphoreType.DMA,
        ],
    )
    return f(y, p1, p2)


def kernel(x, router_w, gate_w, up_w, down_w):
    shape = x.shape
    x_flat = x.reshape(-1, shape[-1]).astype(jnp.float32)

    w1x, w2x, p1, p2, bexp = _router(x_flat, router_w)
    p1 = p1.reshape(-1)
    p2 = p2.reshape(-1)
    xg, wrow = _dispatch(x_flat, w1x, w2x, p1, p2)
    y = _grouped_mm(bexp.reshape(-1)[:NB + 1], xg, gate_w, up_w, down_w,
                    wrow)
    out = _combine(y, p1, p2)
    return out.reshape(shape)


# speedup vs baseline: 1.9153x; 1.0573x over previous
import jax
import jax.numpy as jnp
from jax import lax
from jax.experimental import pallas as pl
from jax.experimental.pallas import tpu as pltpu
from jax.experimental.pallas import tpu_sc as plsc

E = 8
D_MODEL = 1024
D_FF = 2048
T = 2048
BK = 256
BK_BITS = 8
C = 6144
NB = C // BK
NBP = 32
TB = 256
NSTEP = T // TB
NC, NS, L = 2, 16, 16
NW = NC * NS
TPW = T // NW


def _router_body(x_ref, rw_ref, w1x_ref, w2x_ref, p1_ref, p2_ref, bexp_ref,
                 acc_ref, e1s, e2s, r1s, r2s):
    i = pl.program_id(0)

    @pl.when(i == 0)
    def _():
        acc_ref[...] = jnp.zeros_like(acc_ref)

    @pl.when(i < NSTEP)
    def _():
        logits = lax.dot_general(
            x_ref[...], rw_ref[...], (((1,), (1,)), ((), ())),
            preferred_element_type=jnp.float32)
        lane8 = lax.broadcasted_iota(jnp.int32, (TB, E), 1)
        l1 = jnp.max(logits, axis=-1, keepdims=True)
        i1 = jnp.min(jnp.where(logits == l1, lane8, E), axis=-1,
                     keepdims=True)
        neg = jnp.where(lane8 == i1, -jnp.inf, logits)
        l2 = jnp.max(neg, axis=-1, keepdims=True)
        i2 = jnp.min(jnp.where(neg == l2, lane8, E), axis=-1, keepdims=True)
        w1 = 1.0 / (1.0 + jnp.exp(l2 - l1))

        lane16 = lax.broadcasted_iota(jnp.int32, (TB, 16), 1)
        mask = ((lane16 == i1) | (lane16 == i2)).astype(jnp.float32)

        row = lax.broadcasted_iota(jnp.int32, (TB, TB), 0)
        col = lax.broadcasted_iota(jnp.int32, (TB, TB), 1)
        tri = (col < row).astype(jnp.float32)
        ranks = acc_ref[...] + lax.dot_general(
            tri, mask, (((1,), (0,)), ((), ())),
            preferred_element_type=jnp.float32)
        acc_ref[...] = acc_ref[...] + jnp.sum(mask, axis=0, keepdims=True)

        r8 = ranks[:, :E]
        m1 = (lane8 == i1).astype(jnp.float32)
        m2 = (lane8 == i2).astype(jnp.float32)
        r1 = jnp.sum(m1 * r8, axis=-1, keepdims=True).astype(jnp.int32)
        r2 = jnp.sum(m2 * r8, axis=-1, keepdims=True).astype(jnp.int32)

        zerow = jnp.zeros((TB, 128), jnp.float32)
        w1x_ref[...] = zerow + w1
        w2x_ref[...] = zerow + (1.0 - w1)

        sl = pl.ds(i * TB, TB)
        e1s[sl, :] = i1
        e2s[sl, :] = i2
        r1s[sl, :] = r1
        r2s[sl, :] = r2

    @pl.when(i == NSTEP)
    def _():
        cnt = acc_ref[...].astype(jnp.int32)
        padded = ((cnt + (BK - 1)) >> BK_BITS) << BK_BITS
        lane16 = lax.broadcasted_iota(jnp.int32, (1, 16), 1)
        gs = jnp.zeros((1, 16), jnp.int32)
        for e in range(E - 1):
            gs = gs + jnp.where(lane16 > e, padded[:, e:e + 1],
                                jnp.int32(0))

        lane_t = lax.broadcasted_iota(jnp.int32, (T, 16), 1)
        gs_t = gs + jnp.zeros((T, 16), jnp.int32)
        g1 = jnp.sum(jnp.where(lane_t == e1s[...], gs_t, jnp.int32(0)),
                     axis=-1, keepdims=True)
        g2 = jnp.sum(jnp.where(lane_t == e2s[...], gs_t, jnp.int32(0)),
                     axis=-1, keepdims=True)
        p1_ref[...] = g1 + r1s[...]
        p2_ref[...] = g2 + r2s[...]

        bb = lax.broadcasted_iota(jnp.int32, (1, NBP), 1) * BK
        acc = jnp.zeros((1, NBP), jnp.int32)
        for e in range(E):
            acc = acc + jnp.where(bb >= gs[:, e:e + 1], jnp.int32(1),
                                  jnp.int32(0))
        nact = jnp.sum(padded, axis=-1, keepdims=True) >> BK_BITS
        lanebp = lax.broadcasted_iota(jnp.int32, (1, NBP), 1)
        bexp_ref[...] = jnp.where(lanebp == NB, nact + jnp.zeros(
            (1, NBP), jnp.int32), acc - 1)


def _router(x_flat, router_w):
    out_shapes = (
        jax.ShapeDtypeStruct((T, 128), jnp.float32),
        jax.ShapeDtypeStruct((T, 128), jnp.float32),
        jax.ShapeDtypeStruct((T, 1), jnp.int32),
        jax.ShapeDtypeStruct((T, 1), jnp.int32),
        jax.ShapeDtypeStruct((1, NBP), jnp.int32),
    )
    return pl.pallas_call(
        _router_body,
        grid=(NSTEP + 1,),
        in_specs=[
            pl.BlockSpec((TB, D_MODEL),
                         lambda i: (jnp.minimum(i, NSTEP - 1), 0)),
            pl.BlockSpec((E, D_MODEL), lambda i: (0, 0)),
        ],
        out_specs=(
            pl.BlockSpec((TB, 128), lambda i: (jnp.minimum(i, NSTEP - 1), 0)),
            pl.BlockSpec((TB, 128), lambda i: (jnp.minimum(i, NSTEP - 1), 0)),
            pl.BlockSpec((T, 1), lambda i: (0, 0)),
            pl.BlockSpec((T, 1), lambda i: (0, 0)),
            pl.BlockSpec((1, NBP), lambda i: (0, 0)),
        ),
        out_shape=out_shapes,
        scratch_shapes=[
            pltpu.VMEM((1, 16), jnp.float32),
            pltpu.VMEM((T, 1), jnp.int32),
            pltpu.VMEM((T, 1), jnp.int32),
            pltpu.VMEM((T, 1), jnp.int32),
            pltpu.VMEM((T, 1), jnp.int32),
        ],
        compiler_params=pltpu.CompilerParams(
            dimension_semantics=("arbitrary",)),
    )(x_flat, router_w)


def _dispatch_body(x_hbm, w1x_hbm, w2x_hbm, p1_hbm, p2_hbm,
                   xg_hbm, wrow_hbm,
                   xbuf, wb1, wb2, p1v, p2v, sem):
    wid = lax.axis_index("s") * NC + lax.axis_index("c")
    base = wid * TPW

    pltpu.sync_copy(x_hbm.at[pl.ds(base, TPW)], xbuf)
    pltpu.sync_copy(w1x_hbm.at[pl.ds(base, TPW)], wb1)
    pltpu.sync_copy(w2x_hbm.at[pl.ds(base, TPW)], wb2)
    pltpu.sync_copy(p1_hbm.at[pl.ds(base, TPW)], p1v)
    pltpu.sync_copy(p2_hbm.at[pl.ds(base, TPW)], p2v)

    d1 = pltpu.async_copy(xbuf, xg_hbm.at[p1v], sem)
    d2 = pltpu.async_copy(xbuf, xg_hbm.at[p2v], sem)
    d3 = pltpu.async_copy(wb1, wrow_hbm.at[p1v], sem)
    d4 = pltpu.async_copy(wb2, wrow_hbm.at[p2v], sem)
    d1.wait()
    d2.wait()
    d3.wait()
    d4.wait()


def _dispatch(x_flat, w1x, w2x, p1, p2):
    mesh = plsc.VectorSubcoreMesh(
        core_axis_name="c", subcore_axis_name="s",
        num_cores=NC, num_subcores=NS)
    out_type = (
        jax.ShapeDtypeStruct((C, D_MODEL), jnp.float32),
        jax.ShapeDtypeStruct((C, 128), jnp.float32),
    )
    f = pl.kernel(
        _dispatch_body,
        out_type=out_type,
        mesh=mesh,
        scratch_types=[
            pltpu.VMEM((TPW, D_MODEL), jnp.float32),
            pltpu.VMEM((TPW, 128), jnp.float32),
            pltpu.VMEM((TPW, 128), jnp.float32),
            pltpu.VMEM((TPW,), jnp.int32),
            pltpu.VMEM((TPW,), jnp.int32),
            pltpu.SemaphoreType.DMA,
        ],
    )
    return f(x_flat, w1x, w2x, p1, p2)


def _mm_body(bexp_ref, xg_ref, gw_ref, uw_ref, dw_ref, wr_ref, y_ref):
    b = pl.program_id(0)

    @pl.when(b < bexp_ref[NB])
    def _():
        xb = xg_ref[...]
        g = lax.dot_general(xb, gw_ref[0], (((1,), (1,)), ((), ())),
                            preferred_element_type=jnp.float32)
        u = lax.dot_general(xb, uw_ref[0], (((1,), (1,)), ((), ())),
                            preferred_element_type=jnp.float32)
        h = g * jax.nn.sigmoid(g) * u
        y = lax.dot_general(h, dw_ref[0], (((1,), (1,)), ((), ())),
                            preferred_element_type=jnp.float32)
        y_ref[...] = y * wr_ref[:, :1]


def _grouped_mm(bexp, xg, gate_w, up_w, down_w, wrow):
    grid_spec = pltpu.PrefetchScalarGridSpec(
        num_scalar_prefetch=1,
        grid=(NB,),
        in_specs=[
            pl.BlockSpec((BK, D_MODEL), lambda b, s: (b, 0)),
            pl.BlockSpec((1, D_FF, D_MODEL), lambda b, s: (s[b], 0, 0)),
            pl.BlockSpec((1, D_FF, D_MODEL), lambda b, s: (s[b], 0, 0)),
            pl.BlockSpec((1, D_MODEL, D_FF), lambda b, s: (s[b], 0, 0)),
            pl.BlockSpec((BK, 128), lambda b, s: (b, 0)),
        ],
        out_specs=pl.BlockSpec((BK, D_MODEL), lambda b, s: (b, 0)),
    )
    return pl.pallas_call(
        _mm_body,
        grid_spec=grid_spec,
        out_shape=jax.ShapeDtypeStruct((C, D_MODEL), jnp.float32),
        compiler_params=pltpu.CompilerParams(
            dimension_semantics=("arbitrary",)),
    )(bexp, xg, gate_w, up_w, down_w, wrow)


CHUNK = 16
NCH = TPW // CHUNK


def _combine_body(y_hbm, p1_hbm, p2_hbm, out_hbm,
                  p1v, p2v, b1a, b2a, b1b, b2b, sema, semb):
    wid = lax.axis_index("s") * NC + lax.axis_index("c")
    base = wid * TPW

    pltpu.sync_copy(p1_hbm.at[pl.ds(base, TPW)], p1v)
    pltpu.sync_copy(p2_hbm.at[pl.ds(base, TPW)], p2v)

    b1 = (b1a, b1b)
    b2 = (b2a, b2b)
    sems = (sema, semb)

    def issue(ci):
        par = ci & 1
        sl = pl.ds(ci * CHUNK, CHUNK)
        d1 = pltpu.async_copy(y_hbm.at[p1v[sl]], b1[par], sems[par])
        d2 = pltpu.async_copy(y_hbm.at[p2v[sl]], b2[par], sems[par])
        return d1, d2

    cur = issue(0)
    for ci in range(NCH):
        par = ci & 1
        nxt = issue(ci + 1) if ci + 1 < NCH else None
        cur[0].wait()
        cur[1].wait()

        def row_body(r, _):
            for c in range(D_MODEL // L):
                sl = pl.ds(c * L, L)
                b1[par][r, sl] = b1[par][r, sl] + b2[par][r, sl]
            return 0

        lax.fori_loop(0, CHUNK, row_body, 0)
        pltpu.sync_copy(b1[par], out_hbm.at[pl.ds(base + ci * CHUNK, CHUNK)])
        cur = nxt


def _combine(y, p1, p2):
    mesh = plsc.VectorSubcoreMesh(
        core_axis_name="c", subcore_axis_name="s",
        num_cores=NC, num_subcores=NS)
    f = pl.kernel(
        _combine_body,
        out_type=jax.ShapeDtypeStruct((T, D_MODEL), jnp.float32),
        mesh=mesh,
        scratch_types=[
            pltpu.VMEM((TPW,), jnp.int32),
            pltpu.VMEM((TPW,), jnp.int32),
            pltpu.VMEM((CHUNK, D_MODEL), jnp.float32),
            pltpu.VMEM((CHUNK, D_MODEL), jnp.float32),
            pltpu.VMEM((CHUNK, D_MODEL), jnp.float32),
            pltpu.VMEM((CHUNK, D_MODEL), jnp.float32),
            pltpu.SemaphoreType.DMA,
            pltpu.SemaphoreType.DMA,
        ],
    )
    return f(y, p1, p2)


def kernel(x, router_w, gate_w, up_w, down_w):
    shape = x.shape
    x_flat = x.reshape(-1, shape[-1]).astype(jnp.float32)

    w1x, w2x, p1, p2, bexp = _router(x_flat, router_w)
    p1 = p1.reshape(-1)
    p2 = p2.reshape(-1)
    xg, wrow = _dispatch(x_flat, w1x, w2x, p1, p2)
    y = _grouped_mm(bexp.reshape(-1)[:NB + 1], xg, gate_w, up_w, down_w,
                    wrow)
    out = _combine(y, p1, p2)
    return out.reshape(shape)
